# Initial kernel scaffold; baseline (speedup 1.0000x reference)
import jax
import jax.numpy as jnp
from jax.experimental import pallas as pl


def kernel(x, edge_index_r0, edge_index_r1, edge_index_r2, weight, h_bias):
    raise NotImplementedError("write your pallas kernel here")


# speedup vs baseline: 2.9673x; 2.9673x over previous
import jax
import jax.numpy as jnp
from jax import lax
from jax.experimental import pallas as pl
from jax.experimental.pallas import tpu as pltpu
from jax.experimental.pallas import tpu_sc as plsc

N = 10000
D = 128
H = D // 2
E = 320000
R = 3

NC = 2
NS = 16
NW = NC * NS
C = 128
T = -(-E // (NW * C))
E_PAD = NW * T * C
ROWS_PER_SUB = 632
N_ACC = ROWS_PER_SUB * NS
DEG_W = 16


def _sc_body(xp_hbm, s00, s01, dp0, s10, s11, dp1, s20, s21, dp2,
             acc_out, deg_out,
             acc_sh, deg_sh, zbuf, zdeg, ones_v, sidx, didx, rows, sem):
    cid = lax.axis_index("c")
    sid = lax.axis_index("s")
    wid = sid * NC + cid

    def _zrow(i, _):
        for j in range(H // 16):
            zbuf[i, pl.ds(j * 16, 16)] = jnp.zeros((16,), jnp.float32)
        return 0
    lax.fori_loop(0, C, _zrow, 0)

    def _zdeg(i, _):
        zdeg[i, :] = jnp.zeros((DEG_W,), jnp.float32)
        return 0
    lax.fori_loop(0, ROWS_PER_SUB, _zdeg, 0)

    def _ones(i, _):
        ones_v[i, :] = jnp.ones((DEG_W,), jnp.float32)
        return 0
    lax.fori_loop(0, C, _ones, 0)

    base_row = sid * ROWS_PER_SUB
    rels = ((s00, dp0), (s10, dp1), (s20, dp2)), ((s01, dp0), (s11, dp1), (s21, dp2))

    for r in range(R):
        for h in range(2):
            src_hbm, dst_hbm = rels[h][r]
            for off in range(0, ROWS_PER_SUB, C):
                nr = min(C, ROWS_PER_SUB - off)
                pltpu.sync_copy(zbuf.at[pl.ds(0, nr)],
                                acc_sh.at[pl.ds(base_row + off, nr)])
            if h == 0:
                pltpu.sync_copy(zdeg, deg_sh.at[pl.ds(base_row, ROWS_PER_SUB)])
            plsc.subcore_barrier()

            def _chunk(k, _):
                base = (wid * T + k) * C
                pltpu.sync_copy(src_hbm.at[pl.ds(base, C)], sidx)
                pltpu.sync_copy(dst_hbm.at[pl.ds(base, C)], didx)
                pltpu.async_copy(xp_hbm.at[sidx], rows, sem).wait()
                pltpu.sync_copy(rows, acc_sh.at[didx], add=True)
                if h == 0:
                    pltpu.sync_copy(ones_v, deg_sh.at[didx], add=True)
                return 0
            lax.fori_loop(0, T, _chunk, 0)
            plsc.subcore_barrier()

            pltpu.sync_copy(
                acc_sh.at[pl.ds(base_row, ROWS_PER_SUB)],
                acc_out.at[r, cid, h, pl.ds(base_row, ROWS_PER_SUB)])
            if h == 0:
                pltpu.sync_copy(
                    deg_sh.at[pl.ds(base_row, ROWS_PER_SUB)],
                    deg_out.at[r, cid, pl.ds(base_row, ROWS_PER_SUB)])


def _sc_aggregate(xp, args):
    mesh = plsc.VectorSubcoreMesh(core_axis_name="c", subcore_axis_name="s",
                                  num_cores=NC, num_subcores=NS)
    return pl.kernel(
        _sc_body,
        out_type=[
            jax.ShapeDtypeStruct((R, NC, 2, N_ACC, H), jnp.float32),
            jax.ShapeDtypeStruct((R, NC, N_ACC, DEG_W), jnp.float32),
        ],
        mesh=mesh,
        scratch_types=[
            pltpu.MemorySpace.VMEM_SHARED((N_ACC, H), jnp.float32),
            pltpu.MemorySpace.VMEM_SHARED((N_ACC, DEG_W), jnp.float32),
            pltpu.MemorySpace.VMEM((C, H), jnp.float32),
            pltpu.MemorySpace.VMEM((ROWS_PER_SUB, DEG_W), jnp.float32),
            pltpu.MemorySpace.VMEM((C, DEG_W), jnp.float32),
            pltpu.MemorySpace.VMEM((C,), jnp.int32),
            pltpu.MemorySpace.VMEM((C,), jnp.int32),
            pltpu.MemorySpace.VMEM((C, H), jnp.float32),
            pltpu.SemaphoreType.DMA,
        ],
        compiler_params=pltpu.CompilerParams(use_tc_tiling_on_sc=False),
    )(xp, *args)


BN = 1000


def _tc_body(acc_ref, deg_ref, w_ref, b_ref, out_ref):
    a = acc_ref[...]
    s = a[:, 0] + a[:, 1]
    dg = deg_ref[:, 0, :, 0] + deg_ref[:, 1, :, 0]
    dg = jnp.maximum(dg, 1.0)
    o = jnp.broadcast_to(b_ref[...], (BN, D))
    for r in range(R):
        agg = jnp.concatenate([s[r, 0], s[r, 1]], axis=-1)
        o = o + jnp.dot(agg / dg[r][:, None], w_ref[r],
                        preferred_element_type=jnp.float32)
    out_ref[...] = o


def _tc_combine(acc, deg, weight, bias2d):
    return pl.pallas_call(
        _tc_body,
        grid=(N // BN,),
        in_specs=[
            pl.BlockSpec((R, NC, 2, BN, H), lambda i: (0, 0, 0, i, 0)),
            pl.BlockSpec((R, NC, BN, DEG_W), lambda i: (0, 0, i, 0)),
            pl.BlockSpec((R, D, D), lambda i: (0, 0, 0)),
            pl.BlockSpec((1, D), lambda i: (0, 0)),
        ],
        out_specs=pl.BlockSpec((BN, D), lambda i: (i, 0)),
        out_shape=jax.ShapeDtypeStruct((N, D), jnp.float32),
    )(acc, deg, weight, bias2d)


def kernel(x, edge_index_r0, edge_index_r1, edge_index_r2, weight, h_bias):
    xp = x.reshape(2 * N, H)
    pad = E_PAD - E
    args = []
    for ei in (edge_index_r0, edge_index_r1, edge_index_r2):
        src2 = 2 * jnp.concatenate([ei[0], jnp.zeros((pad,), jnp.int32)])
        dstp = jnp.concatenate([ei[1], jnp.full((pad,), N, jnp.int32)])
        args += [src2, src2 + 1, dstp]
    acc, deg = _sc_aggregate(xp, args)
    return _tc_combine(acc, deg, weight, h_bias.reshape(1, D))


# speedup vs baseline: 3.2978x; 1.1114x over previous
import jax
import jax.numpy as jnp
from jax import lax
from jax.experimental import pallas as pl
from jax.experimental.pallas import tpu as pltpu
from jax.experimental.pallas import tpu_sc as plsc

N = 10000
D = 128
H = D // 2
E = 320000
R = 3

NC = 2
NS = 16
NW = NC * NS
C = 128
NBUF = 4
T = 80
G = T // NBUF
E_PAD = NW * T * C
ROWS_PER_SUB = 632
N_ACC = ROWS_PER_SUB * NS
DEG_W = 16


def _sc_body(xp_hbm, s00, s01, dp0, s10, s11, dp1, s20, s21, dp2,
             acc_out, deg_out,
             acc_sh, deg_sh, zbuf, zdeg, ones_v, sidx, didx, rows,
             sem0, sem1, sem2, sem3):
    sems = (sem0, sem1, sem2, sem3)
    cid = lax.axis_index("c")
    sid = lax.axis_index("s")
    wid = sid * NC + cid

    def _zrow(i, _):
        for j in range(H // 16):
            zbuf[i, pl.ds(j * 16, 16)] = jnp.zeros((16,), jnp.float32)
        return 0
    lax.fori_loop(0, C, _zrow, 0)

    def _zdeg(i, _):
        zdeg[i, :] = jnp.zeros((DEG_W,), jnp.float32)
        return 0
    lax.fori_loop(0, ROWS_PER_SUB, _zdeg, 0)

    def _ones(i, _):
        ones_v[i, :] = jnp.ones((DEG_W,), jnp.float32)
        return 0
    lax.fori_loop(0, C, _ones, 0)

    base_row = sid * ROWS_PER_SUB
    rels = ((s00, dp0), (s10, dp1), (s20, dp2)), ((s01, dp0), (s11, dp1), (s21, dp2))

    for r in range(R):
        for h in range(2):
            src_hbm, dst_hbm = rels[h][r]
            for off in range(0, ROWS_PER_SUB, C):
                nr = min(C, ROWS_PER_SUB - off)
                pltpu.sync_copy(zbuf.at[pl.ds(0, nr)],
                                acc_sh.at[pl.ds(base_row + off, nr)])
            if h == 0:
                pltpu.sync_copy(zdeg, deg_sh.at[pl.ds(base_row, ROWS_PER_SUB)])
                pltpu.sync_copy(dst_hbm.at[pl.ds(wid * T, T)], didx)
            pltpu.sync_copy(src_hbm.at[pl.ds(wid * T, T)], sidx)
            for j in range(NBUF):
                pltpu.async_copy(xp_hbm.at[sidx.at[j]], rows.at[j], sems[j])
            plsc.subcore_barrier()

            def _group(g, _):
                for j in range(NBUF):
                    k = g * NBUF + j
                    pltpu.make_async_copy(xp_hbm.at[pl.ds(0, C)],
                                          rows.at[j], sems[j]).wait()
                    k2 = k + NBUF

                    @pl.when(k2 < T)
                    def _():
                        pltpu.async_copy(xp_hbm.at[sidx.at[k2]],
                                         rows.at[j], sems[j])
                    pltpu.sync_copy(rows.at[j], acc_sh.at[didx.at[k]],
                                    add=True)
                    if h == 0:
                        pltpu.sync_copy(ones_v, deg_sh.at[didx.at[k]],
                                        add=True)
                return 0
            lax.fori_loop(0, G, _group, 0)
            plsc.subcore_barrier()

            pltpu.sync_copy(
                acc_sh.at[pl.ds(base_row, ROWS_PER_SUB)],
                acc_out.at[r, cid, h, pl.ds(base_row, ROWS_PER_SUB)])
            if h == 0:
                pltpu.sync_copy(
                    deg_sh.at[pl.ds(base_row, ROWS_PER_SUB)],
                    deg_out.at[r, cid, pl.ds(base_row, ROWS_PER_SUB)])


def _sc_aggregate(xp, args):
    mesh = plsc.VectorSubcoreMesh(core_axis_name="c", subcore_axis_name="s",
                                  num_cores=NC, num_subcores=NS)
    return pl.kernel(
        _sc_body,
        out_type=[
            jax.ShapeDtypeStruct((R, NC, 2, N_ACC, H), jnp.float32),
            jax.ShapeDtypeStruct((R, NC, N_ACC, DEG_W), jnp.float32),
        ],
        mesh=mesh,
        scratch_types=[
            pltpu.MemorySpace.VMEM_SHARED((N_ACC, H), jnp.float32),
            pltpu.MemorySpace.VMEM_SHARED((N_ACC, DEG_W), jnp.float32),
            pltpu.MemorySpace.VMEM((C, H), jnp.float32),
            pltpu.MemorySpace.VMEM((ROWS_PER_SUB, DEG_W), jnp.float32),
            pltpu.MemorySpace.VMEM((C, DEG_W), jnp.float32),
            pltpu.MemorySpace.VMEM((T, C), jnp.int32),
            pltpu.MemorySpace.VMEM((T, C), jnp.int32),
            pltpu.MemorySpace.VMEM((NBUF, C, H), jnp.float32),
            pltpu.SemaphoreType.DMA,
            pltpu.SemaphoreType.DMA,
            pltpu.SemaphoreType.DMA,
            pltpu.SemaphoreType.DMA,
        ],
        compiler_params=pltpu.CompilerParams(use_tc_tiling_on_sc=False),
    )(xp, *args)


BN = 1000


def _tc_body(acc_ref, deg_ref, w_ref, b_ref, out_ref):
    a = acc_ref[...]
    s = a[:, 0] + a[:, 1]
    dg = deg_ref[:, 0, :, 0] + deg_ref[:, 1, :, 0]
    dg = jnp.maximum(dg, 1.0)
    o = jnp.broadcast_to(b_ref[...], (BN, D))
    for r in range(R):
        agg = jnp.concatenate([s[r, 0], s[r, 1]], axis=-1)
        o = o + jnp.dot(agg / dg[r][:, None], w_ref[r],
                        preferred_element_type=jnp.float32)
    out_ref[...] = o


def _tc_combine(acc, deg, weight, bias2d):
    return pl.pallas_call(
        _tc_body,
        grid=(N // BN,),
        in_specs=[
            pl.BlockSpec((R, NC, 2, BN, H), lambda i: (0, 0, 0, i, 0)),
            pl.BlockSpec((R, NC, BN, DEG_W), lambda i: (0, 0, i, 0)),
            pl.BlockSpec((R, D, D), lambda i: (0, 0, 0)),
            pl.BlockSpec((1, D), lambda i: (0, 0)),
        ],
        out_specs=pl.BlockSpec((BN, D), lambda i: (i, 0)),
        out_shape=jax.ShapeDtypeStruct((N, D), jnp.float32),
    )(acc, deg, weight, bias2d)


def kernel(x, edge_index_r0, edge_index_r1, edge_index_r2, weight, h_bias):
    xp = x.reshape(2 * N, H)
    pad = E_PAD - E
    trash = (N + jnp.arange(pad, dtype=jnp.int32) % (N_ACC - N))
    args = []
    for ei in (edge_index_r0, edge_index_r1, edge_index_r2):
        src2 = 2 * jnp.concatenate([ei[0], jnp.zeros((pad,), jnp.int32)])
        dstp = jnp.concatenate([ei[1], trash])
        args += [src2.reshape(NW * T, C), (src2 + 1).reshape(NW * T, C),
                 dstp.reshape(NW * T, C)]
    acc, deg = _sc_aggregate(xp, args)
    return _tc_combine(acc, deg, weight, h_bias.reshape(1, D))
